# baseline (speedup 1.0000x reference)
import functools

import jax
import jax.numpy as jnp
from jax.experimental import pallas as pl
from jax.experimental.pallas import tpu as pltpu


def _fused_kernel(num_levels, cc, nbuf,
                  f_hbm, m_hbm, o_ref, buf, mraw, sel, sem, msem):
    L = num_levels
    B, I, HW = mraw.shape
    KC = o_ref.shape[1]
    C = KC * cc
    T = B * KC

    f4 = f_hbm.reshape(L, B, C, HW)
    m3 = m_hbm.reshape(B, I, HW)

    pltpu.make_async_copy(m3, mraw, msem).start()

    def issue(t):
        b = t // KC
        kc = t % KC
        slot = jax.lax.rem(t, nbuf)
        pltpu.make_async_copy(
            f4.at[:, b, pl.ds(kc * cc, cc), :], buf.at[slot], sem.at[slot]
        ).start()

    for t in range(min(nbuf, T)):
        issue(t)

    pltpu.make_async_copy(m3, mraw, msem).wait()
    m = mraw[...]
    s01 = (m > 0.5).astype(jnp.float32)
    counts = jnp.sum(s01, axis=-1, keepdims=True)
    iota = jax.lax.broadcasted_iota(jnp.int32, m.shape, 2)
    maxv = jnp.max(m, axis=-1, keepdims=True)
    amax = jnp.min(jnp.where(m == maxv, iota, HW), axis=-1, keepdims=True)
    onehot = (iota == amax).astype(jnp.float32)
    sel_eff = jnp.where(counts > 0.0, s01, onehot)
    sel[...] = sel_eff * (1.0 / (L * jnp.maximum(counts, 1.0)))

    def body(t, carry):
        b = t // KC
        kc = t % KC
        slot = jax.lax.rem(t, nbuf)
        pltpu.make_async_copy(
            f4.at[:, b, pl.ds(kc * cc, cc), :], buf.at[slot], sem.at[slot]
        ).wait()
        f = buf[slot, 0]
        for l in range(1, L):
            f = f + buf[slot, l]
        part = jax.lax.dot_general(
            sel[b], f, (((1,), (1,)), ((), ())),
            preferred_element_type=jnp.float32,
        )
        o_ref[b, kc] = part

        @pl.when(t + nbuf < T)
        def _reissue():
            issue(t + nbuf)

        return carry

    jax.lax.fori_loop(0, T, body, None)


def kernel(features, scribbles, batched_fg_coords_list, batched_bg_coords_list,
           random_bg_queries):
    L, B, C, H, W = features.shape
    I = scribbles.shape[1]
    HW = H * W

    cc = 16
    nbuf = 12
    out = pl.pallas_call(
        functools.partial(_fused_kernel, L, cc, nbuf),
        in_specs=[
            pl.BlockSpec(memory_space=pltpu.MemorySpace.HBM),
            pl.BlockSpec(memory_space=pltpu.MemorySpace.HBM),
        ],
        out_specs=pl.BlockSpec(memory_space=pltpu.MemorySpace.VMEM),
        out_shape=jax.ShapeDtypeStruct((B, C // cc, I, cc), jnp.float32),
        scratch_shapes=[
            pltpu.VMEM((nbuf, L, cc, HW), jnp.float32),
            pltpu.VMEM((B, I, HW), jnp.float32),
            pltpu.VMEM((B, I, HW), jnp.float32),
            pltpu.SemaphoreType.DMA((nbuf,)),
            pltpu.SemaphoreType.DMA,
        ],
    )(features, scribbles.astype(jnp.float32))

    out = jnp.transpose(out, (0, 2, 1, 3)).reshape(B, I, C)
    return out[:, None, :, :]


# speedup vs baseline: 1.0067x; 1.0067x over previous
import functools

import jax
import jax.numpy as jnp
from jax.experimental import pallas as pl
from jax.experimental.pallas import tpu as pltpu


def _fused_kernel(num_levels, cc, nbuf,
                  f_hbm, m_hbm, o_ref, buf, mraw, sel, sem, msem):
    L = num_levels
    B, I, HW = mraw.shape
    KC = o_ref.shape[1]
    C = KC * cc
    T = B * KC

    f4 = f_hbm.reshape(L, B, C, HW)
    m3 = m_hbm.reshape(B, I, HW)

    pltpu.make_async_copy(m3, mraw, msem).start()

    def issue(t):
        b = t // KC
        kc = t % KC
        slot = jax.lax.rem(t, nbuf)
        pltpu.make_async_copy(
            f4.at[:, b, pl.ds(kc * cc, cc), :], buf.at[slot], sem.at[slot]
        ).start()

    for t in range(min(nbuf, T)):
        issue(t)

    pltpu.make_async_copy(m3, mraw, msem).wait()
    m = mraw[...]
    s01 = (m > 0.5).astype(jnp.float32)
    counts = jnp.sum(s01, axis=-1, keepdims=True)
    iota = jax.lax.broadcasted_iota(jnp.int32, m.shape, 2)
    maxv = jnp.max(m, axis=-1, keepdims=True)
    amax = jnp.min(jnp.where(m == maxv, iota, HW), axis=-1, keepdims=True)
    onehot = (iota == amax).astype(jnp.float32)
    sel_eff = jnp.where(counts > 0.0, s01, onehot)
    sel[...] = sel_eff * (1.0 / (L * jnp.maximum(counts, 1.0)))

    def body(t, carry):
        b = t // KC
        kc = t % KC
        slot = jax.lax.rem(t, nbuf)
        pltpu.make_async_copy(
            f4.at[:, b, pl.ds(kc * cc, cc), :], buf.at[slot], sem.at[slot]
        ).wait()
        f = buf[slot, 0]
        for l in range(1, L):
            f = f + buf[slot, l]
        part = jax.lax.dot_general(
            f, sel[b], (((1,), (1,)), ((), ())),
            preferred_element_type=jnp.float32,
        )
        o_ref[b, kc] = part

        @pl.when(t + nbuf < T)
        def _reissue():
            issue(t + nbuf)

        return carry

    jax.lax.fori_loop(0, T, body, None)


def kernel(features, scribbles, batched_fg_coords_list, batched_bg_coords_list,
           random_bg_queries):
    L, B, C, H, W = features.shape
    I = scribbles.shape[1]
    HW = H * W

    cc = 16
    nbuf = 12
    out = pl.pallas_call(
        functools.partial(_fused_kernel, L, cc, nbuf),
        in_specs=[
            pl.BlockSpec(memory_space=pltpu.MemorySpace.HBM),
            pl.BlockSpec(memory_space=pltpu.MemorySpace.HBM),
        ],
        out_specs=pl.BlockSpec(memory_space=pltpu.MemorySpace.VMEM),
        out_shape=jax.ShapeDtypeStruct((B, C // cc, cc, I), jnp.float32),
        scratch_shapes=[
            pltpu.VMEM((nbuf, L, cc, HW), jnp.float32),
            pltpu.VMEM((B, I, HW), jnp.float32),
            pltpu.VMEM((B, I, HW), jnp.float32),
            pltpu.SemaphoreType.DMA((nbuf,)),
            pltpu.SemaphoreType.DMA,
        ],
    )(features, scribbles.astype(jnp.float32))

    out = jnp.transpose(out, (0, 3, 1, 2)).reshape(B, I, C)
    return out[:, None, :, :]
